# baseline (speedup 1.0000x reference)
import jax
import jax.numpy as jnp
from jax.experimental import pallas as pl
from jax.experimental.pallas import tpu as pltpu

EMB = 128
N_TAPS = 4
FEAT = 100
FEAT3 = 300
HID = 300
HP = 384
HGRU = 256
TL = 50
TT = 100
WINDOW_SIZES = (2, 3, 4)


def _conv_kernel(tok_ref, tab_hbm, w_ref, scale_ref, shift_ref, madd_ref,
                 o_ref, tab_ref, x_sc, copy_sem):
    tn, lp = tok_ref.shape

    @pl.when(pl.program_id(0) == 0)
    def _():
        pltpu.make_async_copy(tab_hbm, tab_ref, copy_sem).start()
        pltpu.make_async_copy(tab_hbm, tab_ref, copy_sem).wait()

    def gather_sentence(s, carry):
        for jb in range(lp // 8):
            rows = [tab_ref[tok_ref[s, jb * 8 + u], 0] for u in range(8)]
            dest = pl.multiple_of(s * lp + jb * 8, 8)
            x_sc[pl.ds(dest, 8), :] = jnp.stack(rows, axis=0)
        return carry

    jax.lax.fori_loop(0, tn, gather_sentence, 0)

    xf = x_sc[...]
    x = xf.astype(jnp.bfloat16).reshape(tn, lp, EMB)
    xs1 = pltpu.roll(xf, tn * lp - 1, axis=0).astype(
        jnp.bfloat16).reshape(tn, lp, EMB)
    xs2 = pltpu.roll(x, lp - 2, axis=1)
    xs3 = pltpu.roll(xs1, lp - 2, axis=1)

    def tap(v, k):
        return jax.lax.dot_general(
            v, w_ref[k * EMB:(k + 1) * EMB, :],
            dimension_numbers=(((2,), (0,)), ((), ())),
            preferred_element_type=jnp.float32)

    acc = ((tap(x, 0) + tap(xs1, 1)) + (tap(xs2, 2) + tap(xs3, 3)))
    madd = madd_ref[...][None]
    amax = jnp.max(acc + madd, axis=1)
    amin = jnp.min(acc - madd, axis=1)
    s = scale_ref[...]
    best = jnp.where(s > 0.0, s * amax, s * amin) + shift_ref[...]
    o_ref[...] = jnp.maximum(best, 0.0).astype(jnp.bfloat16)


def _conv_features(tokens, table, conv_w, conv_scale, conv_shift):
    n, l = tokens.shape
    v = table.shape[0]
    tile = 64
    l_pad = pl.cdiv(l + N_TAPS - 1, 8) * 8
    n_pad = pl.cdiv(n, tile) * tile
    tokens = jnp.pad(tokens, ((0, n_pad - n), (0, l_pad - l)))
    tab32 = table.astype(jnp.float32).reshape(v, 1, EMB)

    pos = jnp.arange(l_pad, dtype=jnp.int32)[:, None]
    cut = jnp.repeat(jnp.asarray([l - h for h in WINDOW_SIZES], jnp.int32),
                     FEAT)[None, :]
    madd = jnp.where(pos <= cut, 0.0, -1e30).astype(jnp.float32)

    out = pl.pallas_call(
        _conv_kernel,
        out_shape=jax.ShapeDtypeStruct((n_pad, FEAT3), jnp.bfloat16),
        grid=(n_pad // tile,),
        in_specs=[
            pl.BlockSpec((tile, l_pad), lambda i: (i, 0),
                         memory_space=pltpu.SMEM),
            pl.BlockSpec(memory_space=pltpu.MemorySpace.HBM),
            pl.BlockSpec((N_TAPS * EMB, FEAT3), lambda i: (0, 0)),
            pl.BlockSpec((1, FEAT3), lambda i: (0, 0)),
            pl.BlockSpec((1, FEAT3), lambda i: (0, 0)),
            pl.BlockSpec((l_pad, FEAT3), lambda i: (0, 0)),
        ],
        out_specs=pl.BlockSpec((tile, FEAT3), lambda i: (i, 0)),
        scratch_shapes=[pltpu.VMEM((v, 1, EMB), jnp.float32),
                        pltpu.VMEM((tile * l_pad, EMB), jnp.float32),
                        pltpu.SemaphoreType.DMA],
        compiler_params=pltpu.CompilerParams(
            dimension_semantics=("parallel",),
            vmem_limit_bytes=58 * 1024 * 1024),
    )(tokens, tab32, conv_w, conv_scale, conv_shift, madd)
    return out[:n]


def _lstm_kernel(x_ref, wih_ref, whh_ref, b_ref, o_ref, xw_sc, h_sc, c_sc):
    h_sc[...] = jnp.zeros_like(h_sc)
    c_sc[...] = jnp.zeros_like(c_sc)
    tt = x_ref.shape[1]
    xw_sc[...] = jax.lax.dot_general(
        x_ref[0], wih_ref[...],
        dimension_numbers=(((2,), (0,)), ((), ())),
        preferred_element_type=jnp.float32) + b_ref[...]
    whh = whh_ref[...]

    def step(t, carry):
        h = h_sc[...]
        gates = xw_sc[t] + jnp.dot(h.astype(jnp.bfloat16), whh,
                                   preferred_element_type=jnp.float32)
        i = jax.nn.sigmoid(gates[:, 0 * HP:1 * HP])
        f = jax.nn.sigmoid(gates[:, 1 * HP:2 * HP])
        g = jnp.tanh(gates[:, 2 * HP:3 * HP])
        o = jax.nn.sigmoid(gates[:, 3 * HP:4 * HP])
        c_new = f * c_sc[...] + i * g
        h_new = o * jnp.tanh(c_new)
        h_sc[...] = h_new
        c_sc[...] = c_new
        o_ref[0, t] = h_new[:, :HID].astype(jnp.bfloat16)
        return carry

    jax.lax.fori_loop(0, tt, step, 0)


def _lstm_forward(x_seq, wih, whh, b):
    _, t, p, d = x_seq.shape
    return pl.pallas_call(
        _lstm_kernel,
        out_shape=jax.ShapeDtypeStruct((2, t, p, HID), jnp.bfloat16),
        grid=(2,),
        in_specs=[
            pl.BlockSpec((1, t, p, d), lambda i: (i, 0, 0, 0)),
            pl.BlockSpec((d, 4 * HP), lambda i: (0, 0)),
            pl.BlockSpec((HP, 4 * HP), lambda i: (0, 0)),
            pl.BlockSpec((1, 4 * HP), lambda i: (0, 0)),
        ],
        out_specs=pl.BlockSpec((1, t, p, HID), lambda i: (i, 0, 0, 0)),
        scratch_shapes=[pltpu.VMEM((t, p, 4 * HP), jnp.float32),
                        pltpu.VMEM((p, HP), jnp.float32),
                        pltpu.VMEM((p, HP), jnp.float32)],
        compiler_params=pltpu.CompilerParams(
            dimension_semantics=("parallel",)),
    )(x_seq, wih, whh, b)


def _gru_kernel(x_ref, ewih_ref, ewhh_ref, ebih_ref, ebhh_ref,
                dwih_ref, dwhh_ref, dbih_ref, dbhh_ref,
                o_ref, gx_sc, he_sc, hd_sc):
    @pl.when(pl.program_id(0) == 0)
    def _():
        he_sc[...] = jnp.zeros_like(he_sc)
        hd_sc[...] = jnp.zeros_like(hd_sc)

    tt = x_ref.shape[0]
    gx_sc[...] = jax.lax.dot_general(
        x_ref[...], ewih_ref[...],
        dimension_numbers=(((2,), (0,)), ((), ())),
        preferred_element_type=jnp.float32) + ebih_ref[...]
    ewhh = ewhh_ref[...]
    ebhh = ebhh_ref[...]
    dwih = dwih_ref[...]
    dwhh = dwhh_ref[...]
    dbih = dbih_ref[...]
    dbhh = dbhh_ref[...]

    he = he_sc[...]
    hd = hd_sc[...]
    e1, e2 = HGRU, 2 * HGRU
    d1, d2 = HP, 2 * HP
    for s in range(tt):
        gx = gx_sc[s]
        gh = jnp.dot(he.astype(jnp.bfloat16), ewhh,
                     preferred_element_type=jnp.float32) + ebhh
        r = jax.nn.sigmoid(gx[:, :e1] + gh[:, :e1])
        z = jax.nn.sigmoid(gx[:, e1:e2] + gh[:, e1:e2])
        nn = jnp.tanh(gx[:, e2:] + r * gh[:, e2:])
        he = (1.0 - z) * nn + z * he
        heb = he.astype(jnp.bfloat16)
        gxd = jnp.dot(heb, dwih, preferred_element_type=jnp.float32) + dbih
        ghd = jnp.dot(hd.astype(jnp.bfloat16), dwhh,
                      preferred_element_type=jnp.float32) + dbhh
        rd = jax.nn.sigmoid(gxd[:, :d1] + ghd[:, :d1])
        zd = jax.nn.sigmoid(gxd[:, d1:d2] + ghd[:, d1:d2])
        nd = jnp.tanh(gxd[:, d2:] + rd * ghd[:, d2:])
        hd = (1.0 - zd) * nd + zd * hd
        o_ref[s] = hd[:, :HID]
    he_sc[...] = he
    hd_sc[...] = hd


def _gru_encdec(x_seq, ewih, ewhh, ebih, ebhh, dwih, dwhh, dbih, dbhh,
                t_chunk=10):
    t, b, d = x_seq.shape
    return pl.pallas_call(
        _gru_kernel,
        out_shape=jax.ShapeDtypeStruct((t, b, HID), jnp.float32),
        grid=(t // t_chunk,),
        in_specs=[
            pl.BlockSpec((t_chunk, b, d), lambda i: (i, 0, 0)),
            pl.BlockSpec((d, 3 * HGRU), lambda i: (0, 0)),
            pl.BlockSpec((HGRU, 3 * HGRU), lambda i: (0, 0)),
            pl.BlockSpec((1, 3 * HGRU), lambda i: (0, 0)),
            pl.BlockSpec((1, 3 * HGRU), lambda i: (0, 0)),
            pl.BlockSpec((HGRU, 3 * HP), lambda i: (0, 0)),
            pl.BlockSpec((HP, 3 * HP), lambda i: (0, 0)),
            pl.BlockSpec((1, 3 * HP), lambda i: (0, 0)),
            pl.BlockSpec((1, 3 * HP), lambda i: (0, 0)),
        ],
        out_specs=pl.BlockSpec((t_chunk, b, HID), lambda i: (i, 0, 0)),
        scratch_shapes=[pltpu.VMEM((t_chunk, b, 3 * HGRU), jnp.float32),
                        pltpu.VMEM((b, HGRU), jnp.float32),
                        pltpu.VMEM((b, HP), jnp.float32)],
        compiler_params=pltpu.CompilerParams(
            dimension_semantics=("arbitrary",)),
    )(x_seq, ewih, ewhh, ebih, ebhh, dwih, dwhh, dbih, dbhh)


def _pad_gate_cols(w, ngates, h, hp):
    lead = w.shape[0]
    w3 = w.reshape(lead, ngates, h)
    w3 = jnp.pad(w3, ((0, 0), (0, 0), (0, hp - h)))
    return w3.reshape(lead, ngates * hp)


def kernel(embedding, conv_w, conv_scale, conv_shift,
           lstm_wih, lstm_whh, lstm_b,
           enc_wih, enc_whh, enc_bih, enc_bhh,
           dec_wih, dec_whh, dec_bih, dec_bhh,
           fc1_w, fc1_b, fc2_w, fc2_b, fc3_w, fc3_b,
           old, new):
    b = old.shape[0]

    tokens = jnp.concatenate([old, new], axis=0).reshape(2 * b * TL, TT)

    con = _conv_features(tokens, embedding, conv_w, conv_scale,
                         conv_shift)
    con = con.reshape(2, b, TL, FEAT3)

    wih_p = _pad_gate_cols(lstm_wih, 4, HID, HP)
    whh_p = jnp.pad(_pad_gate_cols(lstm_whh, 4, HID, HP),
                    ((0, HP - HID), (0, 0)))
    b_p = _pad_gate_cols(lstm_b, 4, HID, HP)
    lstm_out = _lstm_forward(con, wih_p, whh_p, b_p)

    fuse = jnp.maximum(lstm_out[0], lstm_out[1])
    xf = jnp.transpose(fuse, (1, 0, 2))

    dwih_p = _pad_gate_cols(dec_wih, 3, HID, HP)
    dwhh_p = jnp.pad(_pad_gate_cols(dec_whh, 3, HID, HP),
                     ((0, HP - HID), (0, 0)))
    dbih_p = _pad_gate_cols(dec_bih, 3, HID, HP)
    dbhh_p = _pad_gate_cols(dec_bhh, 3, HID, HP)
    dec = _gru_encdec(xf, enc_wih, enc_whh, enc_bih, enc_bhh,
                      dwih_p, dwhh_p, dbih_p, dbhh_p)
    de = jnp.transpose(dec, (1, 0, 2))

    pooled = jnp.max(de.reshape(b, TL, 10, 30), axis=-1).reshape(b, 500)
    h1 = jnp.dot(pooled, fc1_w) + fc1_b
    h2 = jnp.dot(h1, fc2_w) + fc2_b
    return jnp.dot(h2, fc3_w) + fc3_b


# speedup vs baseline: 1.0166x; 1.0166x over previous
import jax
import jax.numpy as jnp
from jax.experimental import pallas as pl
from jax.experimental.pallas import tpu as pltpu

EMB = 128
N_TAPS = 4
FEAT = 100
FEAT3 = 300
HID = 300
HP = 384
HGRU = 256
TL = 50
TT = 100
WINDOW_SIZES = (2, 3, 4)


def _conv_kernel(tok_ref, tab_hbm, w_ref, scale_ref, shift_ref, madd_ref,
                 o_ref, tab_ref, x_sc, copy_sem):
    tn, lp = tok_ref.shape

    @pl.when(pl.program_id(0) == 0)
    def _():
        pltpu.make_async_copy(tab_hbm, tab_ref, copy_sem).start()
        pltpu.make_async_copy(tab_hbm, tab_ref, copy_sem).wait()

    def gather_pair(p, carry):
        for si in range(2):
            s = 2 * p + si
            for jb in range(lp // 8):
                rows = [tab_ref[tok_ref[s, jb * 8 + u], 0] for u in range(8)]
                dest = pl.multiple_of(s * lp + jb * 8, 8)
                x_sc[pl.ds(dest, 8), :] = jnp.stack(rows, axis=0)
        return carry

    jax.lax.fori_loop(0, tn // 2, gather_pair, 0)

    x = x_sc[...].astype(jnp.bfloat16).reshape(tn, lp, EMB)
    xs1 = pltpu.roll(x, lp - 1, axis=1)
    xs2 = pltpu.roll(x, lp - 2, axis=1)
    xs3 = pltpu.roll(xs1, lp - 2, axis=1)

    def tap(v, k):
        return jax.lax.dot_general(
            v, w_ref[k * EMB:(k + 1) * EMB, :],
            dimension_numbers=(((2,), (0,)), ((), ())),
            preferred_element_type=jnp.float32)

    acc = ((tap(x, 0) + tap(xs1, 1)) + (tap(xs2, 2) + tap(xs3, 3)))
    madd = madd_ref[...][None]
    amax = jnp.max(acc + madd, axis=1)
    amin = jnp.min(acc - madd, axis=1)
    s = scale_ref[...]
    best = jnp.where(s > 0.0, s * amax, s * amin) + shift_ref[...]
    o_ref[...] = jnp.maximum(best, 0.0).astype(jnp.bfloat16)


def _conv_features(tokens, table, conv_w, conv_scale, conv_shift):
    n, l = tokens.shape
    v = table.shape[0]
    tile = 64
    l_pad = pl.cdiv(l + N_TAPS - 1, 8) * 8
    n_pad = pl.cdiv(n, tile) * tile
    tokens = jnp.pad(tokens, ((0, n_pad - n), (0, l_pad - l)))
    tab32 = table.astype(jnp.float32).reshape(v, 1, EMB)

    pos = jnp.arange(l_pad, dtype=jnp.int32)[:, None]
    cut = jnp.repeat(jnp.asarray([l - h for h in WINDOW_SIZES], jnp.int32),
                     FEAT)[None, :]
    madd = jnp.where(pos <= cut, 0.0, -1e30).astype(jnp.float32)

    out = pl.pallas_call(
        _conv_kernel,
        out_shape=jax.ShapeDtypeStruct((n_pad, FEAT3), jnp.bfloat16),
        grid=(n_pad // tile,),
        in_specs=[
            pl.BlockSpec((tile, l_pad), lambda i: (i, 0),
                         memory_space=pltpu.SMEM),
            pl.BlockSpec(memory_space=pltpu.MemorySpace.HBM),
            pl.BlockSpec((N_TAPS * EMB, FEAT3), lambda i: (0, 0)),
            pl.BlockSpec((1, FEAT3), lambda i: (0, 0)),
            pl.BlockSpec((1, FEAT3), lambda i: (0, 0)),
            pl.BlockSpec((l_pad, FEAT3), lambda i: (0, 0)),
        ],
        out_specs=pl.BlockSpec((tile, FEAT3), lambda i: (i, 0)),
        scratch_shapes=[pltpu.VMEM((v, 1, EMB), jnp.float32),
                        pltpu.VMEM((tile * l_pad, EMB), jnp.float32),
                        pltpu.SemaphoreType.DMA],
        compiler_params=pltpu.CompilerParams(
            dimension_semantics=("parallel",),
            vmem_limit_bytes=58 * 1024 * 1024),
    )(tokens, tab32, conv_w, conv_scale, conv_shift, madd)
    return out[:n]


def _lstm_kernel(x_ref, wih_ref, whh_ref, b_ref, o_ref, xw_sc, h_sc, c_sc):
    h_sc[...] = jnp.zeros_like(h_sc)
    c_sc[...] = jnp.zeros_like(c_sc)
    tt = x_ref.shape[1]
    xw_sc[...] = jax.lax.dot_general(
        x_ref[0], wih_ref[...],
        dimension_numbers=(((2,), (0,)), ((), ())),
        preferred_element_type=jnp.float32) + b_ref[...]
    whh = whh_ref[...]

    def step(t, carry):
        h = h_sc[...]
        gates = xw_sc[t] + jnp.dot(h.astype(jnp.bfloat16), whh,
                                   preferred_element_type=jnp.float32)
        i = jax.nn.sigmoid(gates[:, 0 * HP:1 * HP])
        f = jax.nn.sigmoid(gates[:, 1 * HP:2 * HP])
        g = jnp.tanh(gates[:, 2 * HP:3 * HP])
        o = jax.nn.sigmoid(gates[:, 3 * HP:4 * HP])
        c_new = f * c_sc[...] + i * g
        h_new = o * jnp.tanh(c_new)
        h_sc[...] = h_new
        c_sc[...] = c_new
        o_ref[0, t] = h_new[:, :HID].astype(jnp.bfloat16)
        return carry

    jax.lax.fori_loop(0, tt, step, 0)


def _lstm_forward(x_seq, wih, whh, b):
    _, t, p, d = x_seq.shape
    return pl.pallas_call(
        _lstm_kernel,
        out_shape=jax.ShapeDtypeStruct((2, t, p, HID), jnp.bfloat16),
        grid=(2,),
        in_specs=[
            pl.BlockSpec((1, t, p, d), lambda i: (i, 0, 0, 0)),
            pl.BlockSpec((d, 4 * HP), lambda i: (0, 0)),
            pl.BlockSpec((HP, 4 * HP), lambda i: (0, 0)),
            pl.BlockSpec((1, 4 * HP), lambda i: (0, 0)),
        ],
        out_specs=pl.BlockSpec((1, t, p, HID), lambda i: (i, 0, 0, 0)),
        scratch_shapes=[pltpu.VMEM((t, p, 4 * HP), jnp.float32),
                        pltpu.VMEM((p, HP), jnp.float32),
                        pltpu.VMEM((p, HP), jnp.float32)],
        compiler_params=pltpu.CompilerParams(
            dimension_semantics=("parallel",)),
    )(x_seq, wih, whh, b)


def _gru_kernel(x_ref, ewih_ref, ewhh_ref, ebih_ref, ebhh_ref,
                dwih_ref, dwhh_ref, dbih_ref, dbhh_ref,
                o_ref, gx_sc, he_sc, hd_sc):
    @pl.when(pl.program_id(0) == 0)
    def _():
        he_sc[...] = jnp.zeros_like(he_sc)
        hd_sc[...] = jnp.zeros_like(hd_sc)

    tt = x_ref.shape[0]
    gx_sc[...] = jax.lax.dot_general(
        x_ref[...], ewih_ref[...],
        dimension_numbers=(((2,), (0,)), ((), ())),
        preferred_element_type=jnp.float32) + ebih_ref[...]
    ewhh = ewhh_ref[...]
    ebhh = ebhh_ref[...]
    dwih = dwih_ref[...]
    dwhh = dwhh_ref[...]
    dbih = dbih_ref[...]
    dbhh = dbhh_ref[...]

    he = he_sc[...]
    hd = hd_sc[...]
    e1, e2 = HGRU, 2 * HGRU
    d1, d2 = HP, 2 * HP
    for s in range(tt):
        gx = gx_sc[s]
        gh = jnp.dot(he.astype(jnp.bfloat16), ewhh,
                     preferred_element_type=jnp.float32) + ebhh
        r = jax.nn.sigmoid(gx[:, :e1] + gh[:, :e1])
        z = jax.nn.sigmoid(gx[:, e1:e2] + gh[:, e1:e2])
        nn = jnp.tanh(gx[:, e2:] + r * gh[:, e2:])
        he = (1.0 - z) * nn + z * he
        heb = he.astype(jnp.bfloat16)
        gxd = jnp.dot(heb, dwih, preferred_element_type=jnp.float32) + dbih
        ghd = jnp.dot(hd.astype(jnp.bfloat16), dwhh,
                      preferred_element_type=jnp.float32) + dbhh
        rd = jax.nn.sigmoid(gxd[:, :d1] + ghd[:, :d1])
        zd = jax.nn.sigmoid(gxd[:, d1:d2] + ghd[:, d1:d2])
        nd = jnp.tanh(gxd[:, d2:] + rd * ghd[:, d2:])
        hd = (1.0 - zd) * nd + zd * hd
        o_ref[s] = hd[:, :HID]
    he_sc[...] = he
    hd_sc[...] = hd


def _gru_encdec(x_seq, ewih, ewhh, ebih, ebhh, dwih, dwhh, dbih, dbhh,
                t_chunk=10):
    t, b, d = x_seq.shape
    return pl.pallas_call(
        _gru_kernel,
        out_shape=jax.ShapeDtypeStruct((t, b, HID), jnp.float32),
        grid=(t // t_chunk,),
        in_specs=[
            pl.BlockSpec((t_chunk, b, d), lambda i: (i, 0, 0)),
            pl.BlockSpec((d, 3 * HGRU), lambda i: (0, 0)),
            pl.BlockSpec((HGRU, 3 * HGRU), lambda i: (0, 0)),
            pl.BlockSpec((1, 3 * HGRU), lambda i: (0, 0)),
            pl.BlockSpec((1, 3 * HGRU), lambda i: (0, 0)),
            pl.BlockSpec((HGRU, 3 * HP), lambda i: (0, 0)),
            pl.BlockSpec((HP, 3 * HP), lambda i: (0, 0)),
            pl.BlockSpec((1, 3 * HP), lambda i: (0, 0)),
            pl.BlockSpec((1, 3 * HP), lambda i: (0, 0)),
        ],
        out_specs=pl.BlockSpec((t_chunk, b, HID), lambda i: (i, 0, 0)),
        scratch_shapes=[pltpu.VMEM((t_chunk, b, 3 * HGRU), jnp.float32),
                        pltpu.VMEM((b, HGRU), jnp.float32),
                        pltpu.VMEM((b, HP), jnp.float32)],
        compiler_params=pltpu.CompilerParams(
            dimension_semantics=("arbitrary",)),
    )(x_seq, ewih, ewhh, ebih, ebhh, dwih, dwhh, dbih, dbhh)


def _pad_gate_cols(w, ngates, h, hp):
    lead = w.shape[0]
    w3 = w.reshape(lead, ngates, h)
    w3 = jnp.pad(w3, ((0, 0), (0, 0), (0, hp - h)))
    return w3.reshape(lead, ngates * hp)


def kernel(embedding, conv_w, conv_scale, conv_shift,
           lstm_wih, lstm_whh, lstm_b,
           enc_wih, enc_whh, enc_bih, enc_bhh,
           dec_wih, dec_whh, dec_bih, dec_bhh,
           fc1_w, fc1_b, fc2_w, fc2_b, fc3_w, fc3_b,
           old, new):
    b = old.shape[0]

    tokens = jnp.concatenate([old, new], axis=0).reshape(2 * b * TL, TT)

    con = _conv_features(tokens, embedding, conv_w, conv_scale,
                         conv_shift)
    con = con.reshape(2, b, TL, FEAT3)

    wih_p = _pad_gate_cols(lstm_wih, 4, HID, HP)
    whh_p = jnp.pad(_pad_gate_cols(lstm_whh, 4, HID, HP),
                    ((0, HP - HID), (0, 0)))
    b_p = _pad_gate_cols(lstm_b, 4, HID, HP)
    lstm_out = _lstm_forward(con, wih_p, whh_p, b_p)

    fuse = jnp.maximum(lstm_out[0], lstm_out[1])
    xf = jnp.transpose(fuse, (1, 0, 2))

    dwih_p = _pad_gate_cols(dec_wih, 3, HID, HP)
    dwhh_p = jnp.pad(_pad_gate_cols(dec_whh, 3, HID, HP),
                     ((0, HP - HID), (0, 0)))
    dbih_p = _pad_gate_cols(dec_bih, 3, HID, HP)
    dbhh_p = _pad_gate_cols(dec_bhh, 3, HID, HP)
    dec = _gru_encdec(xf, enc_wih, enc_whh, enc_bih, enc_bhh,
                      dwih_p, dwhh_p, dbih_p, dbhh_p)
    de = jnp.transpose(dec, (1, 0, 2))

    pooled = jnp.max(de.reshape(b, TL, 10, 30), axis=-1).reshape(b, 500)
    h1 = jnp.dot(pooled, fc1_w) + fc1_b
    h2 = jnp.dot(h1, fc2_w) + fc2_b
    return jnp.dot(h2, fc3_w) + fc3_b


# speedup vs baseline: 1.0218x; 1.0051x over previous
import jax
import jax.numpy as jnp
from jax.experimental import pallas as pl
from jax.experimental.pallas import tpu as pltpu

EMB = 128
N_TAPS = 4
FEAT = 100
FEAT3 = 300
HID = 300
HP = 384
HGRU = 256
TL = 50
TT = 100
WINDOW_SIZES = (2, 3, 4)


def _conv_kernel(tok_ref, tab_hbm, w_ref, scale_ref, shift_ref, madd_ref,
                 o_ref, tab_ref, x_sc, copy_sem):
    tn, lp = tok_ref.shape

    @pl.when(pl.program_id(0) == 0)
    def _():
        pltpu.make_async_copy(tab_hbm, tab_ref, copy_sem).start()
        pltpu.make_async_copy(tab_hbm, tab_ref, copy_sem).wait()

    def gather_pair(p, carry):
        for si in range(4):
            s = 4 * p + si
            for jb in range(lp // 8):
                rows = [tab_ref[tok_ref[s, jb * 8 + u], 0] for u in range(8)]
                dest = pl.multiple_of(s * lp + jb * 8, 8)
                x_sc[pl.ds(dest, 8), :] = jnp.stack(rows, axis=0)
        return carry

    jax.lax.fori_loop(0, tn // 4, gather_pair, 0)

    x = x_sc[...].astype(jnp.bfloat16).reshape(tn, lp, EMB)
    xs1 = pltpu.roll(x, lp - 1, axis=1)
    xs2 = pltpu.roll(x, lp - 2, axis=1)
    xs3 = pltpu.roll(xs1, lp - 2, axis=1)

    def tap(v, k):
        return jax.lax.dot_general(
            v, w_ref[k * EMB:(k + 1) * EMB, :],
            dimension_numbers=(((2,), (0,)), ((), ())),
            preferred_element_type=jnp.float32)

    acc = ((tap(x, 0) + tap(xs1, 1)) + (tap(xs2, 2) + tap(xs3, 3)))
    madd = madd_ref[...][None]
    amax = jnp.max(acc + madd, axis=1)
    amin = jnp.min(acc - madd, axis=1)
    s = scale_ref[...]
    best = jnp.where(s > 0.0, s * amax, s * amin) + shift_ref[...]
    o_ref[...] = jnp.maximum(best, 0.0).astype(jnp.bfloat16)


def _conv_features(tokens, table, conv_w, conv_scale, conv_shift):
    n, l = tokens.shape
    v = table.shape[0]
    tile = 64
    l_pad = pl.cdiv(l + N_TAPS - 1, 8) * 8
    n_pad = pl.cdiv(n, tile) * tile
    tokens = jnp.pad(tokens, ((0, n_pad - n), (0, l_pad - l)))
    tab32 = table.astype(jnp.float32).reshape(v, 1, EMB)

    pos = jnp.arange(l_pad, dtype=jnp.int32)[:, None]
    cut = jnp.repeat(jnp.asarray([l - h for h in WINDOW_SIZES], jnp.int32),
                     FEAT)[None, :]
    madd = jnp.where(pos <= cut, 0.0, -1e30).astype(jnp.float32)

    out = pl.pallas_call(
        _conv_kernel,
        out_shape=jax.ShapeDtypeStruct((n_pad, FEAT3), jnp.bfloat16),
        grid=(n_pad // tile,),
        in_specs=[
            pl.BlockSpec((tile, l_pad), lambda i: (i, 0),
                         memory_space=pltpu.SMEM),
            pl.BlockSpec(memory_space=pltpu.MemorySpace.HBM),
            pl.BlockSpec((N_TAPS * EMB, FEAT3), lambda i: (0, 0)),
            pl.BlockSpec((1, FEAT3), lambda i: (0, 0)),
            pl.BlockSpec((1, FEAT3), lambda i: (0, 0)),
            pl.BlockSpec((l_pad, FEAT3), lambda i: (0, 0)),
        ],
        out_specs=pl.BlockSpec((tile, FEAT3), lambda i: (i, 0)),
        scratch_shapes=[pltpu.VMEM((v, 1, EMB), jnp.float32),
                        pltpu.VMEM((tile * l_pad, EMB), jnp.float32),
                        pltpu.SemaphoreType.DMA],
        compiler_params=pltpu.CompilerParams(
            dimension_semantics=("parallel",),
            vmem_limit_bytes=58 * 1024 * 1024),
    )(tokens, tab32, conv_w, conv_scale, conv_shift, madd)
    return out[:n]


def _lstm_kernel(x_ref, wih_ref, whh_ref, b_ref, o_ref, xw_sc, h_sc, c_sc):
    h_sc[...] = jnp.zeros_like(h_sc)
    c_sc[...] = jnp.zeros_like(c_sc)
    tt = x_ref.shape[1]
    xw_sc[...] = jax.lax.dot_general(
        x_ref[0], wih_ref[...],
        dimension_numbers=(((2,), (0,)), ((), ())),
        preferred_element_type=jnp.float32) + b_ref[...]
    whh = whh_ref[...]

    def step(t, carry):
        h = h_sc[...]
        gates = xw_sc[t] + jnp.dot(h.astype(jnp.bfloat16), whh,
                                   preferred_element_type=jnp.float32)
        i = jax.nn.sigmoid(gates[:, 0 * HP:1 * HP])
        f = jax.nn.sigmoid(gates[:, 1 * HP:2 * HP])
        g = jnp.tanh(gates[:, 2 * HP:3 * HP])
        o = jax.nn.sigmoid(gates[:, 3 * HP:4 * HP])
        c_new = f * c_sc[...] + i * g
        h_new = o * jnp.tanh(c_new)
        h_sc[...] = h_new
        c_sc[...] = c_new
        o_ref[0, t] = h_new[:, :HID].astype(jnp.bfloat16)
        return carry

    jax.lax.fori_loop(0, tt, step, 0)


def _lstm_forward(x_seq, wih, whh, b):
    _, t, p, d = x_seq.shape
    return pl.pallas_call(
        _lstm_kernel,
        out_shape=jax.ShapeDtypeStruct((2, t, p, HID), jnp.bfloat16),
        grid=(2,),
        in_specs=[
            pl.BlockSpec((1, t, p, d), lambda i: (i, 0, 0, 0)),
            pl.BlockSpec((d, 4 * HP), lambda i: (0, 0)),
            pl.BlockSpec((HP, 4 * HP), lambda i: (0, 0)),
            pl.BlockSpec((1, 4 * HP), lambda i: (0, 0)),
        ],
        out_specs=pl.BlockSpec((1, t, p, HID), lambda i: (i, 0, 0, 0)),
        scratch_shapes=[pltpu.VMEM((t, p, 4 * HP), jnp.float32),
                        pltpu.VMEM((p, HP), jnp.float32),
                        pltpu.VMEM((p, HP), jnp.float32)],
        compiler_params=pltpu.CompilerParams(
            dimension_semantics=("parallel",)),
    )(x_seq, wih, whh, b)


def _gru_kernel(x_ref, ewih_ref, ewhh_ref, ebih_ref, ebhh_ref,
                dwih_ref, dwhh_ref, dbih_ref, dbhh_ref,
                o_ref, gx_sc, he_sc, hd_sc):
    @pl.when(pl.program_id(0) == 0)
    def _():
        he_sc[...] = jnp.zeros_like(he_sc)
        hd_sc[...] = jnp.zeros_like(hd_sc)

    tt = x_ref.shape[0]
    gx_sc[...] = jax.lax.dot_general(
        x_ref[...], ewih_ref[...],
        dimension_numbers=(((2,), (0,)), ((), ())),
        preferred_element_type=jnp.float32) + ebih_ref[...]
    ewhh = ewhh_ref[...]
    ebhh = ebhh_ref[...]
    dwih = dwih_ref[...]
    dwhh = dwhh_ref[...]
    dbih = dbih_ref[...]
    dbhh = dbhh_ref[...]

    he = he_sc[...]
    hd = hd_sc[...]
    e1, e2 = HGRU, 2 * HGRU
    d1, d2 = HP, 2 * HP
    for s in range(tt):
        gx = gx_sc[s]
        gh = jnp.dot(he.astype(jnp.bfloat16), ewhh,
                     preferred_element_type=jnp.float32) + ebhh
        r = jax.nn.sigmoid(gx[:, :e1] + gh[:, :e1])
        z = jax.nn.sigmoid(gx[:, e1:e2] + gh[:, e1:e2])
        nn = jnp.tanh(gx[:, e2:] + r * gh[:, e2:])
        he = (1.0 - z) * nn + z * he
        heb = he.astype(jnp.bfloat16)
        gxd = jnp.dot(heb, dwih, preferred_element_type=jnp.float32) + dbih
        ghd = jnp.dot(hd.astype(jnp.bfloat16), dwhh,
                      preferred_element_type=jnp.float32) + dbhh
        rd = jax.nn.sigmoid(gxd[:, :d1] + ghd[:, :d1])
        zd = jax.nn.sigmoid(gxd[:, d1:d2] + ghd[:, d1:d2])
        nd = jnp.tanh(gxd[:, d2:] + rd * ghd[:, d2:])
        hd = (1.0 - zd) * nd + zd * hd
        o_ref[s] = hd[:, :HID]
    he_sc[...] = he
    hd_sc[...] = hd


def _gru_encdec(x_seq, ewih, ewhh, ebih, ebhh, dwih, dwhh, dbih, dbhh,
                t_chunk=10):
    t, b, d = x_seq.shape
    return pl.pallas_call(
        _gru_kernel,
        out_shape=jax.ShapeDtypeStruct((t, b, HID), jnp.float32),
        grid=(t // t_chunk,),
        in_specs=[
            pl.BlockSpec((t_chunk, b, d), lambda i: (i, 0, 0)),
            pl.BlockSpec((d, 3 * HGRU), lambda i: (0, 0)),
            pl.BlockSpec((HGRU, 3 * HGRU), lambda i: (0, 0)),
            pl.BlockSpec((1, 3 * HGRU), lambda i: (0, 0)),
            pl.BlockSpec((1, 3 * HGRU), lambda i: (0, 0)),
            pl.BlockSpec((HGRU, 3 * HP), lambda i: (0, 0)),
            pl.BlockSpec((HP, 3 * HP), lambda i: (0, 0)),
            pl.BlockSpec((1, 3 * HP), lambda i: (0, 0)),
            pl.BlockSpec((1, 3 * HP), lambda i: (0, 0)),
        ],
        out_specs=pl.BlockSpec((t_chunk, b, HID), lambda i: (i, 0, 0)),
        scratch_shapes=[pltpu.VMEM((t_chunk, b, 3 * HGRU), jnp.float32),
                        pltpu.VMEM((b, HGRU), jnp.float32),
                        pltpu.VMEM((b, HP), jnp.float32)],
        compiler_params=pltpu.CompilerParams(
            dimension_semantics=("arbitrary",)),
    )(x_seq, ewih, ewhh, ebih, ebhh, dwih, dwhh, dbih, dbhh)


def _pad_gate_cols(w, ngates, h, hp):
    lead = w.shape[0]
    w3 = w.reshape(lead, ngates, h)
    w3 = jnp.pad(w3, ((0, 0), (0, 0), (0, hp - h)))
    return w3.reshape(lead, ngates * hp)


def kernel(embedding, conv_w, conv_scale, conv_shift,
           lstm_wih, lstm_whh, lstm_b,
           enc_wih, enc_whh, enc_bih, enc_bhh,
           dec_wih, dec_whh, dec_bih, dec_bhh,
           fc1_w, fc1_b, fc2_w, fc2_b, fc3_w, fc3_b,
           old, new):
    b = old.shape[0]

    tokens = jnp.concatenate([old, new], axis=0).reshape(2 * b * TL, TT)

    con = _conv_features(tokens, embedding, conv_w, conv_scale,
                         conv_shift)
    con = con.reshape(2, b, TL, FEAT3)

    wih_p = _pad_gate_cols(lstm_wih, 4, HID, HP)
    whh_p = jnp.pad(_pad_gate_cols(lstm_whh, 4, HID, HP),
                    ((0, HP - HID), (0, 0)))
    b_p = _pad_gate_cols(lstm_b, 4, HID, HP)
    lstm_out = _lstm_forward(con, wih_p, whh_p, b_p)

    fuse = jnp.maximum(lstm_out[0], lstm_out[1])
    xf = jnp.transpose(fuse, (1, 0, 2))

    dwih_p = _pad_gate_cols(dec_wih, 3, HID, HP)
    dwhh_p = jnp.pad(_pad_gate_cols(dec_whh, 3, HID, HP),
                     ((0, HP - HID), (0, 0)))
    dbih_p = _pad_gate_cols(dec_bih, 3, HID, HP)
    dbhh_p = _pad_gate_cols(dec_bhh, 3, HID, HP)
    dec = _gru_encdec(xf, enc_wih, enc_whh, enc_bih, enc_bhh,
                      dwih_p, dwhh_p, dbih_p, dbhh_p)
    de = jnp.transpose(dec, (1, 0, 2))

    pooled = jnp.max(de.reshape(b, TL, 10, 30), axis=-1).reshape(b, 500)
    h1 = jnp.dot(pooled, fc1_w) + fc1_b
    h2 = jnp.dot(h1, fc2_w) + fc2_b
    return jnp.dot(h2, fc3_w) + fc3_b
